# baseline (speedup 1.0000x reference)
import functools

import jax
import jax.numpy as jnp
from jax import lax
from jax.experimental import pallas as pl
from jax.experimental.pallas import tpu as pltpu
from jax.experimental.pallas import tpu_sc as plsc

ROWS = 8192
DIM = 128
NUM_CORES = 2
ROWS_PER_CORE = ROWS // NUM_CORES
NUM_CHUNKS = 2
CHUNK_ROWS = ROWS_PER_CORE // NUM_CHUNKS

_mesh = plsc.ScalarSubcoreMesh(axis_name="c", num_cores=NUM_CORES)


@functools.partial(
    pl.kernel,
    mesh=_mesh,
    out_type=jax.ShapeDtypeStruct((ROWS, DIM), jnp.float32),
    scratch_types=[
        pltpu.VMEM_SHARED((2, CHUNK_ROWS, DIM), jnp.float32),
        pltpu.SemaphoreType.DMA,
        pltpu.SemaphoreType.DMA,
        pltpu.SemaphoreType.DMA,
        pltpu.SemaphoreType.DMA,
    ],
)
def _pos_embed_lookup(table_hbm, out_hbm, buf_s, rs0, rs1, ws0, ws1):
    cid = lax.axis_index("c")
    base = cid * ROWS_PER_CORE
    rsem = (rs0, rs1)
    wsem = (ws0, ws1)
    reads = [None] * NUM_CHUNKS
    writes = [None] * NUM_CHUNKS

    reads[0] = pltpu.async_copy(
        table_hbm.at[pl.ds(base, CHUNK_ROWS)], buf_s.at[0], rsem[0]
    )
    for i in range(NUM_CHUNKS):
        b = i % 2
        if i + 1 < NUM_CHUNKS:
            nb = (i + 1) % 2
            if i >= 1:
                writes[i - 1].wait()
            reads[i + 1] = pltpu.async_copy(
                table_hbm.at[pl.ds(base + (i + 1) * CHUNK_ROWS, CHUNK_ROWS)],
                buf_s.at[nb],
                rsem[nb],
            )
        reads[i].wait()
        writes[i] = pltpu.async_copy(
            buf_s.at[b],
            out_hbm.at[pl.ds(base + i * CHUNK_ROWS, CHUNK_ROWS)],
            wsem[b],
        )
    for i in range(max(0, NUM_CHUNKS - 2), NUM_CHUNKS):
        writes[i].wait()


def kernel(x, pos_table):
    del x
    return _pos_embed_lookup(pos_table)


# speedup vs baseline: 1.0391x; 1.0391x over previous
import functools

import jax
import jax.numpy as jnp
from jax import lax
from jax.experimental import pallas as pl
from jax.experimental.pallas import tpu as pltpu
from jax.experimental.pallas import tpu_sc as plsc

ROWS = 8192
DIM = 128
NUM_CORES = 2
NUM_SUBCORES = 16
NUM_WORKERS = NUM_CORES * NUM_SUBCORES
ROWS_PER_WORKER = ROWS // NUM_WORKERS

_mesh = plsc.VectorSubcoreMesh(core_axis_name="c", subcore_axis_name="s")


@functools.partial(
    pl.kernel,
    mesh=_mesh,
    out_type=jax.ShapeDtypeStruct((ROWS, DIM), jnp.float32),
    scratch_types=[pltpu.VMEM((ROWS_PER_WORKER, DIM), jnp.float32)],
)
def _pos_embed_lookup(table_hbm, out_hbm, buf_v):
    wid = lax.axis_index("s") * NUM_CORES + lax.axis_index("c")
    base = wid * ROWS_PER_WORKER
    pltpu.sync_copy(table_hbm.at[pl.ds(base, ROWS_PER_WORKER)], buf_v)
    pltpu.sync_copy(buf_v, out_hbm.at[pl.ds(base, ROWS_PER_WORKER)])


def kernel(x, pos_table):
    del x
    return _pos_embed_lookup(pos_table)


# speedup vs baseline: 1.0407x; 1.0016x over previous
import functools

import jax
import jax.numpy as jnp
from jax import lax
from jax.experimental import pallas as pl
from jax.experimental.pallas import tpu as pltpu
from jax.experimental.pallas import tpu_sc as plsc

ROWS = 8192
DIM = 128
NUM_CORES = 2
NUM_SUBCORES = 16
NUM_WORKERS = NUM_CORES * NUM_SUBCORES
ROWS_PER_WORKER = ROWS // NUM_WORKERS

_mesh = plsc.VectorSubcoreMesh(core_axis_name="c", subcore_axis_name="s")


HALF_ROWS = ROWS_PER_WORKER // 2


@functools.partial(
    pl.kernel,
    mesh=_mesh,
    out_type=jax.ShapeDtypeStruct((ROWS, DIM), jnp.float32),
    scratch_types=[
        pltpu.VMEM((ROWS_PER_WORKER, DIM), jnp.float32),
        pltpu.SemaphoreType.DMA,
        pltpu.SemaphoreType.DMA,
        pltpu.SemaphoreType.DMA,
        pltpu.SemaphoreType.DMA,
    ],
)
def _pos_embed_lookup(table_hbm, out_hbm, buf_v, rs0, rs1, ws0, ws1):
    wid = lax.axis_index("s") * NUM_CORES + lax.axis_index("c")
    base = wid * ROWS_PER_WORKER
    r0 = pltpu.async_copy(
        table_hbm.at[pl.ds(base, HALF_ROWS)], buf_v.at[pl.ds(0, HALF_ROWS)], rs0
    )
    r1 = pltpu.async_copy(
        table_hbm.at[pl.ds(base + HALF_ROWS, HALF_ROWS)],
        buf_v.at[pl.ds(HALF_ROWS, HALF_ROWS)],
        rs1,
    )
    r0.wait()
    w0 = pltpu.async_copy(
        buf_v.at[pl.ds(0, HALF_ROWS)], out_hbm.at[pl.ds(base, HALF_ROWS)], ws0
    )
    r1.wait()
    w1 = pltpu.async_copy(
        buf_v.at[pl.ds(HALF_ROWS, HALF_ROWS)],
        out_hbm.at[pl.ds(base + HALF_ROWS, HALF_ROWS)],
        ws1,
    )
    w0.wait()
    w1.wait()


def kernel(x, pos_table):
    del x
    return _pos_embed_lookup(pos_table)
